# Initial kernel scaffold; baseline (speedup 1.0000x reference)
import jax
import jax.numpy as jnp
from jax.experimental import pallas as pl


def kernel(x, router_logits, w13_weight, w2_weight, top_k, renormalize):
    raise NotImplementedError("write your pallas kernel here")


# speedup vs baseline: 1.2615x; 1.2615x over previous
import functools

import jax
import jax.numpy as jnp
from jax.experimental import pallas as pl
from jax.experimental.pallas import tpu as pltpu


def _moe_kernel(params_ref, x_ref, logits_ref, w13_ref, w2_ref, out_ref,
                ids_ref, ws_ref, *, d_ff, num_experts):
    e = pl.program_id(0)

    @pl.when(e == 0)
    def _routing():
        logits = logits_ref[:]
        t, n_e = logits.shape
        m = jnp.max(logits, axis=-1, keepdims=True)
        ex = jnp.exp(logits - m)
        probs = ex / jnp.sum(ex, axis=-1, keepdims=True)
        eidx = jax.lax.broadcasted_iota(jnp.int32, (t, n_e), 1)
        w1 = jnp.max(probs, axis=-1, keepdims=True)
        a1 = jnp.min(jnp.where(probs == w1, eidx, n_e), axis=-1, keepdims=True)
        masked = jnp.where(eidx == a1, -1.0, probs)
        w2v = jnp.max(masked, axis=-1, keepdims=True)
        a2 = jnp.min(jnp.where(masked == w2v, eidx, n_e), axis=-1,
                     keepdims=True)
        top_k = params_ref[0]
        renorm = params_ref[1]
        w1e = jnp.where(top_k >= 1, w1, 0.0)
        w2e = jnp.where(top_k >= 2, w2v, 0.0)
        denom = w1e + w2e
        w1n = jnp.where(renorm != 0, w1e / denom, w1e)
        w2n = jnp.where(renorm != 0, w2e / denom, w2e)
        ids_ref[:, 0:1] = a1
        ids_ref[:, 1:2] = a2
        ws_ref[:, 0:1] = w1n
        ws_ref[:, 1:2] = w2n
        out_ref[:] = jnp.zeros_like(out_ref)

    w13e = w13_ref[0]
    xv = x_ref[:]
    h = jax.lax.dot_general(xv, w13e, (((1,), (1,)), ((), ())),
                            preferred_element_type=jnp.float32)
    gate = h[:, :d_ff]
    up = h[:, d_ff:]
    act = gate * jax.lax.logistic(gate) * up
    o = jax.lax.dot_general(act, w2_ref[0], (((1,), (1,)), ((), ())),
                            preferred_element_type=jnp.float32)
    c = (jnp.where(ids_ref[:, 0:1] == e, ws_ref[:, 0:1], 0.0)
         + jnp.where(ids_ref[:, 1:2] == e, ws_ref[:, 1:2], 0.0))
    out_ref[:] += c * o


def kernel(x, router_logits, w13_weight, w2_weight, top_k, renormalize):
    t, h = x.shape
    n_e = w13_weight.shape[0]
    d_ff = w13_weight.shape[1] // 2
    params = jnp.stack([jnp.asarray(top_k, jnp.int32),
                        jnp.asarray(renormalize, jnp.int32)])

    body = functools.partial(_moe_kernel, d_ff=d_ff, num_experts=n_e)
    out = pl.pallas_call(
        body,
        grid=(n_e,),
        in_specs=[
            pl.BlockSpec(memory_space=pltpu.SMEM),
            pl.BlockSpec((t, h), lambda e: (0, 0)),
            pl.BlockSpec((t, n_e), lambda e: (0, 0)),
            pl.BlockSpec((1, 2 * d_ff, h), lambda e: (e, 0, 0)),
            pl.BlockSpec((1, h, d_ff), lambda e: (e, 0, 0)),
        ],
        out_specs=pl.BlockSpec((t, h), lambda e: (0, 0)),
        out_shape=jax.ShapeDtypeStruct((t, h), jnp.float32),
        scratch_shapes=[
            pltpu.VMEM((t, 2), jnp.int32),
            pltpu.VMEM((t, 2), jnp.float32),
        ],
        compiler_params=pltpu.CompilerParams(
            dimension_semantics=("arbitrary",),
        ),
    )(params, x, router_logits, w13_weight, w2_weight)
    return out.astype(x.dtype)


# speedup vs baseline: 1.4322x; 1.1353x over previous
import functools

import jax
import jax.numpy as jnp
from jax.experimental import pallas as pl
from jax.experimental.pallas import tpu as pltpu


def _routing_kernel(params_ref, logits_ref, ids_ref, ws_ref, perm_ref,
                    cnt_ref):
    logits = logits_ref[:]
    t, n_e = logits.shape
    m = jnp.max(logits, axis=-1, keepdims=True)
    ex = jnp.exp(logits - m)
    probs = ex / jnp.sum(ex, axis=-1, keepdims=True)
    eidx = jax.lax.broadcasted_iota(jnp.int32, (t, n_e), 1)
    w1 = jnp.max(probs, axis=-1, keepdims=True)
    a1 = jnp.min(jnp.where(probs == w1, eidx, n_e), axis=-1, keepdims=True)
    masked = jnp.where(eidx == a1, -1.0, probs)
    w2v = jnp.max(masked, axis=-1, keepdims=True)
    a2 = jnp.min(jnp.where(masked == w2v, eidx, n_e), axis=-1, keepdims=True)
    top_k = params_ref[0]
    renorm = params_ref[1]
    w1e = jnp.where(top_k >= 1, w1, 0.0)
    w2e = jnp.where(top_k >= 2, w2v, 0.0)
    denom = w1e + w2e
    w1n = jnp.where(renorm != 0, w1e / denom, w1e)
    w2n = jnp.where(renorm != 0, w2e / denom, w2e)
    ids_ref[:, 0:1] = a1
    ids_ref[:, 1:2] = a2
    ws_ref[:, 0:1] = w1n
    ws_ref[:, 1:2] = w2n

    hit1 = jnp.where(top_k >= 1, (a1 == eidx).astype(jnp.float32), 0.0)
    hit2 = jnp.where(top_k >= 2, (a2 == eidx).astype(jnp.float32), 0.0)
    active_row = jnp.max(jnp.maximum(hit1, hit2), axis=0, keepdims=True)

    ii = jax.lax.broadcasted_iota(jnp.int32, (n_e, n_e), 0)
    jj = jax.lax.broadcasted_iota(jnp.int32, (n_e, n_e), 1)
    ltri = (ii < jj).astype(jnp.float32)
    rank_row = jax.lax.dot_general(active_row, ltri, (((1,), (0,)), ((), ())),
                                   preferred_element_type=jnp.float32)
    diag = (ii == jj)
    rank_col = jnp.sum(jnp.where(diag, jnp.broadcast_to(rank_row, (n_e, n_e)),
                                 0.0), axis=1, keepdims=True)
    act_col = jnp.sum(jnp.where(diag, jnp.broadcast_to(active_row,
                                                       (n_e, n_e)),
                                0.0), axis=1, keepdims=True)
    e_col = jax.lax.broadcasted_iota(jnp.int32, (n_e, 1), 0).astype(jnp.float32)
    sel = act_col * jnp.where(rank_col == jj.astype(jnp.float32), 1.0, 0.0)
    perm_row = jnp.sum(e_col * sel, axis=0, keepdims=True)
    perm_ref[:] = perm_row.astype(jnp.int32)
    cnt_ref[:] = jnp.sum(active_row, axis=1, keepdims=True).astype(jnp.int32)


def _ffn_kernel(perm_ref, cnt_ref, ids_ref, ws_ref, x_ref, w13_ref, w2_ref,
                out_ref, *, d_ff):
    i = pl.program_id(0)
    cnt = cnt_ref[0]
    e = perm_ref[jnp.minimum(i, cnt - 1)]

    @pl.when(i == 0)
    def _init():
        out_ref[:] = jnp.zeros_like(out_ref)

    @pl.when(i < cnt)
    def _compute():
        w13e = w13_ref[0]
        xv = x_ref[:]
        h = jax.lax.dot_general(xv, w13e, (((1,), (1,)), ((), ())),
                                preferred_element_type=jnp.float32)
        gate = h[:, :d_ff]
        up = h[:, d_ff:]
        act = gate * jax.lax.logistic(gate) * up
        o = jax.lax.dot_general(act, w2_ref[0], (((1,), (1,)), ((), ())),
                                preferred_element_type=jnp.float32)
        c = (jnp.where(ids_ref[:, 0:1] == e, ws_ref[:, 0:1], 0.0)
             + jnp.where(ids_ref[:, 1:2] == e, ws_ref[:, 1:2], 0.0))
        out_ref[:] += c * o


def kernel(x, router_logits, w13_weight, w2_weight, top_k, renormalize):
    t, h = x.shape
    n_e = w13_weight.shape[0]
    d_ff = w13_weight.shape[1] // 2
    params = jnp.stack([jnp.asarray(top_k, jnp.int32),
                        jnp.asarray(renormalize, jnp.int32)])

    ids, ws, perm2d, cnt2d = pl.pallas_call(
        _routing_kernel,
        in_specs=[
            pl.BlockSpec(memory_space=pltpu.SMEM),
            pl.BlockSpec((t, n_e), lambda: (0, 0)),
        ],
        out_specs=[
            pl.BlockSpec((t, 2), lambda: (0, 0)),
            pl.BlockSpec((t, 2), lambda: (0, 0)),
            pl.BlockSpec((1, n_e), lambda: (0, 0)),
            pl.BlockSpec((1, 1), lambda: (0, 0)),
        ],
        out_shape=[
            jax.ShapeDtypeStruct((t, 2), jnp.int32),
            jax.ShapeDtypeStruct((t, 2), jnp.float32),
            jax.ShapeDtypeStruct((1, n_e), jnp.int32),
            jax.ShapeDtypeStruct((1, 1), jnp.int32),
        ],
    )(params, router_logits)
    perm = perm2d.reshape(n_e)
    cnt = cnt2d.reshape(1)

    def _w13_map(i, perm_ref, cnt_ref):
        return (perm_ref[jnp.minimum(i, cnt_ref[0] - 1)], 0, 0)

    def _w2_map(i, perm_ref, cnt_ref):
        return (perm_ref[jnp.minimum(i, cnt_ref[0] - 1)], 0, 0)

    body = functools.partial(_ffn_kernel, d_ff=d_ff)
    grid_spec = pltpu.PrefetchScalarGridSpec(
        num_scalar_prefetch=2,
        grid=(n_e,),
        in_specs=[
            pl.BlockSpec((t, 2), lambda i, p, c: (0, 0)),
            pl.BlockSpec((t, 2), lambda i, p, c: (0, 0)),
            pl.BlockSpec((t, h), lambda i, p, c: (0, 0)),
            pl.BlockSpec((1, 2 * d_ff, h), _w13_map),
            pl.BlockSpec((1, h, d_ff), _w2_map),
        ],
        out_specs=pl.BlockSpec((t, h), lambda i, p, c: (0, 0)),
    )
    out = pl.pallas_call(
        body,
        grid_spec=grid_spec,
        out_shape=jax.ShapeDtypeStruct((t, h), jnp.float32),
        compiler_params=pltpu.CompilerParams(
            dimension_semantics=("arbitrary",),
        ),
    )(perm, cnt, ids, ws, x, w13_weight, w2_weight)
    return out.astype(x.dtype)
